# Initial kernel scaffold; baseline (speedup 1.0000x reference)
import jax
import jax.numpy as jnp
from jax.experimental import pallas as pl


def kernel(h, x, edge_attr, params, edge_index):
    raise NotImplementedError("write your pallas kernel here")


# speedup vs baseline: 3.0255x; 3.0255x over previous
import functools

import jax
import jax.numpy as jnp
from jax import lax
from jax.experimental import pallas as pl
from jax.experimental.pallas import tpu as pltpu
from jax.experimental.pallas import tpu_sc as plsc

N = 10000
E = 320000
D = 128
NW = 32
EPW = E // NW
GB = 80
NB = EPW // GB
RPT = 640
RC = 80
TE = 2560
TN = 2000
NORM_FACTOR = 100.0

_mesh = plsc.VectorSubcoreMesh(core_axis_name="c", subcore_axis_name="s")


def _worker_id():
    return lax.axis_index("s") * 2 + lax.axis_index("c")


@functools.partial(
    pl.kernel, mesh=_mesh,
    out_type=[
        jax.ShapeDtypeStruct((E, D), jnp.float32),
        jax.ShapeDtypeStruct((E, D), jnp.float32),
        jax.ShapeDtypeStruct((4 * E,), jnp.float32),
    ],
    scratch_types=[
        pltpu.VMEM((GB,), jnp.int32),
        pltpu.VMEM((GB,), jnp.int32),
        pltpu.VMEM((GB, D), jnp.float32),
        pltpu.VMEM((GB, D), jnp.float32),
        pltpu.VMEM((N,), jnp.float32),
        pltpu.VMEM((N,), jnp.float32),
        pltpu.VMEM((N,), jnp.float32),
        pltpu.VMEM((GB,), jnp.float32),
        pltpu.VMEM((GB,), jnp.float32),
        pltpu.VMEM((GB,), jnp.float32),
        pltpu.VMEM((GB,), jnp.float32),
        pltpu.SemaphoreType.DMA,
    ],
    compiler_params=pltpu.CompilerParams(needs_layout_passes=False),
)
def _gather_hx(h_hbm, x0_hbm, x1_hbm, x2_hbm, row_hbm, col_hbm,
               hrow_hbm, hcol_hbm, flat4_hbm,
               idx_r, idx_c, br, bc, x0, x1, x2, sr, s0, s1, s2, sem):
    pltpu.sync_copy(x0_hbm, x0)
    pltpu.sync_copy(x1_hbm, x1)
    pltpu.sync_copy(x2_hbm, x2)
    base = _worker_id() * EPW

    def step(g, carry):
        off = base + g * GB
        pltpu.sync_copy(row_hbm.at[pl.ds(off, GB)], idx_r)
        pltpu.sync_copy(col_hbm.at[pl.ds(off, GB)], idx_c)
        c1 = pltpu.async_copy(h_hbm.at[idx_r], br, sem)
        c2 = pltpu.async_copy(h_hbm.at[idx_c], bc, sem)
        for k in range(GB // 16):
            sl = pl.ds(k * 16, 16)
            ir = idx_r[sl]
            ic = idx_c[sl]
            d0 = plsc.load_gather(x0, [ir]) - plsc.load_gather(x0, [ic])
            d1 = plsc.load_gather(x1, [ir]) - plsc.load_gather(x1, [ic])
            d2 = plsc.load_gather(x2, [ir]) - plsc.load_gather(x2, [ic])
            sr[sl] = d0 * d0 + d1 * d1 + d2 * d2
            s0[sl] = d0
            s1[sl] = d1
            s2[sl] = d2
        c1.wait()
        c2.wait()
        pltpu.sync_copy(br, hrow_hbm.at[pl.ds(off, GB)])
        pltpu.sync_copy(bc, hcol_hbm.at[pl.ds(off, GB)])
        pltpu.sync_copy(sr, flat4_hbm.at[pl.ds(0 * E + off, GB)])
        pltpu.sync_copy(s0, flat4_hbm.at[pl.ds(1 * E + off, GB)])
        pltpu.sync_copy(s1, flat4_hbm.at[pl.ds(2 * E + off, GB)])
        pltpu.sync_copy(s2, flat4_hbm.at[pl.ds(3 * E + off, GB)])
        return carry

    lax.fori_loop(0, NB, step, 0)


@functools.partial(
    pl.kernel, mesh=_mesh,
    out_type=[
        jax.ShapeDtypeStruct((E, D), jnp.float32),
        jax.ShapeDtypeStruct((E, D), jnp.float32),
    ],
    scratch_types=[
        pltpu.VMEM((GB,), jnp.int32),
        pltpu.VMEM((GB,), jnp.int32),
        pltpu.VMEM((GB, D), jnp.float32),
        pltpu.VMEM((GB, D), jnp.float32),
        pltpu.SemaphoreType.DMA,
    ],
)
def _gather_h(h_hbm, row_hbm, col_hbm, hrow_hbm, hcol_hbm,
              idx_r, idx_c, br, bc, sem):
    base = _worker_id() * EPW

    def step(g, carry):
        off = base + g * GB
        pltpu.sync_copy(row_hbm.at[pl.ds(off, GB)], idx_r)
        pltpu.sync_copy(col_hbm.at[pl.ds(off, GB)], idx_c)
        c1 = pltpu.async_copy(h_hbm.at[idx_r], br, sem)
        c2 = pltpu.async_copy(h_hbm.at[idx_c], bc, sem)
        c1.wait()
        c2.wait()
        pltpu.sync_copy(br, hrow_hbm.at[pl.ds(off, GB)])
        pltpu.sync_copy(bc, hcol_hbm.at[pl.ds(off, GB)])
        return carry

    lax.fori_loop(0, NB, step, 0)


@functools.partial(
    pl.kernel, mesh=_mesh,
    out_type=jax.ShapeDtypeStruct((2 * N, D), jnp.float32),
    scratch_types=[
        pltpu.VMEM((GB,), jnp.int32),
        pltpu.VMEM((GB, D), jnp.float32),
        pltpu.VMEM((RC, D), jnp.float32),
        pltpu.VMEM_SHARED((N, D), jnp.float32),
    ],
)
def _scatter_d(feat_hbm, row_hbm, out_hbm, idx_v, feat_v, stage, table):
    cid = lax.axis_index("c")
    sid = lax.axis_index("s")
    wid = sid * 2 + cid
    rbase = sid * RPT
    nchunk = jnp.where(sid == 15, (N - 15 * RPT) // RC, RPT // RC)

    def zrow(i, carry):
        def zcol(j, c2):
            stage[i, pl.ds(j * 16, 16)] = jnp.zeros((16,), jnp.float32)
            return c2
        return lax.fori_loop(0, D // 16, zcol, carry)
    lax.fori_loop(0, RC, zrow, 0)

    def ztab(k, carry):
        pltpu.sync_copy(stage, table.at[pl.ds(rbase + k * RC, RC)])
        return carry
    lax.fori_loop(0, nchunk, ztab, 0)
    plsc.subcore_barrier()

    base = wid * EPW

    def step(g, carry):
        off = base + g * GB
        pltpu.sync_copy(row_hbm.at[pl.ds(off, GB)], idx_v)
        pltpu.sync_copy(feat_hbm.at[pl.ds(off, GB)], feat_v)
        pltpu.sync_copy(feat_v, table.at[idx_v], add=True)
        return carry
    lax.fori_loop(0, NB, step, 0)
    plsc.subcore_barrier()

    def wb(k, carry):
        r = rbase + k * RC
        pltpu.sync_copy(table.at[pl.ds(r, RC)], stage)
        pltpu.sync_copy(stage, out_hbm.at[pl.ds(cid * N + r, RC)])
        return carry
    lax.fori_loop(0, nchunk, wb, 0)


def _edge_body(hr, hc, s, w1r, w1c, m, b1, w2, b2, aw, ab, feat):
    z = hr[...] @ w1r[...] + hc[...] @ w1c[...]
    z = z + lax.dot_general(s[...], m[...], (((0,), (0,)), ((), ())))
    z = jax.nn.silu(z + b1[...])
    mij = jax.nn.silu(z @ w2[...] + b2[...])
    att = jax.nn.sigmoid(mij @ aw[...] + ab[...])
    feat[...] = mij * att


def _equiv_body(hr, hc, s, w1r, w1c, m, b1, w2, b2, w3, wdx, wrad, trans):
    z = hr[...] @ w1r[...] + hc[...] @ w1c[...]
    z = z + lax.dot_general(s[...], m[...], (((0,), (0,)), ((), ())))
    z = jax.nn.silu(z + b1[...])
    z = jax.nn.silu(z @ w2[...] + b2[...])
    phi = z @ w3[...]
    dx = lax.dot_general(s[...], wdx[...], (((0,), (0,)), ((), ())))
    rad = lax.dot_general(s[...], wrad[...], (((0,), (0,)), ((), ())))
    cdn = dx / (jnp.sqrt(rad + 1e-8) + 1.0)
    trans[...] = cdn * phi


def _node_body(h, a0, a1, w1h, w1a, b1, w2, b2, out):
    agg = (a0[...] + a1[...]) * (1.0 / NORM_FACTOR)
    z = jax.nn.silu(h[...] @ w1h[...] + agg @ w1a[...] + b1[...])
    out[...] = h[...] + z @ w2[...] + b2[...]


def _xupd_body(x, p0, p1, out):
    upd = (p0[...] + p1[...])[:, 0:3] * (1.0 / NORM_FACTOR)
    out[...] = x[...] + upd


def _espec(w):
    return pl.BlockSpec((TE, w), lambda i: (i, 0))


def _wspec(r, c):
    return pl.BlockSpec((r, c), lambda i: (0, 0))


def _sspec():
    return pl.BlockSpec((5, TE), lambda i: (0, i))


def _scal_mat(w1e):
    return jnp.concatenate(
        [w1e[0:1], jnp.zeros((3, D), jnp.float32), w1e[1:2]], axis=0)


def _edge_mlp(hrow, hcol, s, p):
    return pl.pallas_call(
        _edge_body,
        grid=(E // TE,),
        in_specs=[_espec(D), _espec(D), _sspec(),
                  _wspec(D, D), _wspec(D, D), _wspec(5, D), _wspec(1, D),
                  _wspec(D, D), _wspec(1, D), _wspec(D, 1), _wspec(1, 1)],
        out_specs=_espec(D),
        out_shape=jax.ShapeDtypeStruct((E, D), jnp.float32),
    )(hrow, hcol, s, p['e_w1'][0:D], p['e_w1'][D:2 * D],
      _scal_mat(p['e_w1'][2 * D:]), p['e_b1'].reshape(1, D),
      p['e_w2'], p['e_b2'].reshape(1, D), p['a_w'], p['a_b'].reshape(1, 1))


def _equiv_mlp(hrow, hcol, s, p):
    wdx = jnp.zeros((5, D), jnp.float32).at[1, 0].set(1.0) \
        .at[2, 1].set(1.0).at[3, 2].set(1.0)
    wrad = jnp.zeros((5, D), jnp.float32).at[0].set(1.0)
    return pl.pallas_call(
        _equiv_body,
        grid=(E // TE,),
        in_specs=[_espec(D), _espec(D), _sspec(),
                  _wspec(D, D), _wspec(D, D), _wspec(5, D), _wspec(1, D),
                  _wspec(D, D), _wspec(1, D), _wspec(D, 1),
                  _wspec(5, D), _wspec(5, D)],
        out_specs=_espec(D),
        out_shape=jax.ShapeDtypeStruct((E, D), jnp.float32),
    )(hrow, hcol, s, p['c_w1'][0:D], p['c_w1'][D:2 * D],
      _scal_mat(p['c_w1'][2 * D:]), p['c_b1'].reshape(1, D),
      p['c_w2'], p['c_b2'].reshape(1, D), p['c_w3'], wdx, wrad)


def _node_mlp(h, parts, p):
    nspec = pl.BlockSpec((TN, D), lambda i: (i, 0))
    return pl.pallas_call(
        _node_body,
        grid=(N // TN,),
        in_specs=[nspec,
                  pl.BlockSpec((TN, D), lambda i: (i, 0)),
                  pl.BlockSpec((TN, D), lambda i: (i + N // TN, 0)),
                  _wspec(D, D), _wspec(D, D), _wspec(1, D),
                  _wspec(D, D), _wspec(1, D)],
        out_specs=nspec,
        out_shape=jax.ShapeDtypeStruct((N, D), jnp.float32),
    )(h, parts, parts, p['n_w1'][0:D], p['n_w1'][D:2 * D],
      p['n_b1'].reshape(1, D), p['n_w2'], p['n_b2'].reshape(1, D))


def _x_update(x, xparts):
    nspec3 = pl.BlockSpec((TN, 3), lambda i: (i, 0))
    return pl.pallas_call(
        _xupd_body,
        grid=(N // TN,),
        in_specs=[nspec3,
                  pl.BlockSpec((TN, D), lambda i: (i, 0)),
                  pl.BlockSpec((TN, D), lambda i: (i + N // TN, 0))],
        out_specs=nspec3,
        out_shape=jax.ShapeDtypeStruct((N, 3), jnp.float32),
    )(x, xparts, xparts)


def kernel(h, x, edge_attr, params, edge_index):
    row = edge_index[0]
    col = edge_index[1]

    hrow, hcol, flat4 = _gather_hx(h, x[:, 0], x[:, 1], x[:, 2], row, col)
    s = jnp.concatenate(
        [flat4.reshape(4, E), edge_attr.reshape(1, E)], axis=0)

    p0 = params['gcl_0']
    feat = _edge_mlp(hrow, hcol, s, p0)
    parts = _scatter_d(feat, row)
    h = _node_mlp(h, parts, p0)

    p1 = params['gcl_1']
    hrow, hcol = _gather_h(h, row, col)
    feat = _edge_mlp(hrow, hcol, s, p1)
    parts = _scatter_d(feat, row)
    h = _node_mlp(h, parts, p1)

    pe = params['equiv']
    hrow, hcol = _gather_h(h, row, col)
    trans = _equiv_mlp(hrow, hcol, s, pe)
    xparts = _scatter_d(trans, row)
    x = _x_update(x, xparts)

    return (h, x)


# speedup vs baseline: 3.7925x; 1.2535x over previous
import functools

import jax
import jax.numpy as jnp
from jax import lax
from jax.experimental import pallas as pl
from jax.experimental.pallas import tpu as pltpu
from jax.experimental.pallas import tpu_sc as plsc

N = 10000
E = 320000
D = 128
NW = 32
EPW = E // NW
CPW = 2 * E // NW
GB = 80
NBG = CPW // GB
NBS = EPW // GB
SB = 2000
RPT = 640
RC = 80
TE = 2560
TN = 2000
NORM_FACTOR = 100.0

_mesh = plsc.VectorSubcoreMesh(core_axis_name="c", subcore_axis_name="s")
_sc_params = pltpu.CompilerParams(needs_layout_passes=False)


def _worker_id():
    return lax.axis_index("s") * 2 + lax.axis_index("c")


def _pipelined_gather(h_hbm, idxcat_hbm, hcat_hbm, idx_all, bufs, sgs, sws,
                      base):
    pltpu.sync_copy(idxcat_hbm.at[pl.ds(base, CPW)], idx_all)
    pltpu.async_copy(h_hbm.at[idx_all.at[pl.ds(0, GB)]], bufs[0], sgs[0])

    @pl.loop(0, NBG, step=2)
    def _loop(g0):
        for b in range(2):
            g = g0 + b
            buf, sg, sw = bufs[b], sgs[b], sws[b]
            obuf, osg, osw = bufs[1 - b], sgs[1 - b], sws[1 - b]
            pltpu.make_async_copy(
                h_hbm.at[idx_all.at[pl.ds(g * GB, GB)]], buf, sg).wait()
            pltpu.async_copy(buf, hcat_hbm.at[pl.ds(base + g * GB, GB)], sw)

            @pl.when(g >= 1)
            def _():
                pltpu.make_async_copy(
                    obuf, hcat_hbm.at[pl.ds(base + (g - 1) * GB, GB)],
                    osw).wait()

            @pl.when(g + 1 < NBG)
            def _():
                pltpu.async_copy(
                    h_hbm.at[idx_all.at[pl.ds((g + 1) * GB, GB)]], obuf, osg)

    last = (NBG - 1) % 2
    pltpu.make_async_copy(
        bufs[last], hcat_hbm.at[pl.ds(base + (NBG - 1) * GB, GB)],
        sws[last]).wait()


@functools.partial(
    pl.kernel, mesh=_mesh,
    out_type=[
        jax.ShapeDtypeStruct((2 * E, D), jnp.float32),
        jax.ShapeDtypeStruct((4 * E,), jnp.float32),
    ],
    scratch_types=[
        pltpu.VMEM((CPW,), jnp.int32),
        pltpu.VMEM((EPW,), jnp.int32),
        pltpu.VMEM((EPW,), jnp.int32),
        pltpu.VMEM((GB, D), jnp.float32),
        pltpu.VMEM((GB, D), jnp.float32),
        pltpu.VMEM((N,), jnp.float32),
        pltpu.VMEM((N,), jnp.float32),
        pltpu.VMEM((N,), jnp.float32),
        pltpu.VMEM((SB,), jnp.float32),
        pltpu.VMEM((SB,), jnp.float32),
        pltpu.VMEM((SB,), jnp.float32),
        pltpu.VMEM((SB,), jnp.float32),
        pltpu.SemaphoreType.DMA,
        pltpu.SemaphoreType.DMA,
        pltpu.SemaphoreType.DMA,
        pltpu.SemaphoreType.DMA,
    ],
    compiler_params=_sc_params,
)
def _gather_hx(h_hbm, x0_hbm, x1_hbm, x2_hbm, idxcat_hbm,
               hcat_hbm, flat4_hbm,
               idx_all, idx_r, idx_c, b0, b1, x0, x1, x2,
               sr, s0, s1, s2, sg0, sg1, sw0, sw1):
    wid = _worker_id()
    pbase = wid * EPW
    pltpu.sync_copy(idxcat_hbm.at[pl.ds(pbase, EPW)], idx_r)
    pltpu.sync_copy(idxcat_hbm.at[pl.ds(E + pbase, EPW)], idx_c)
    pltpu.sync_copy(x0_hbm, x0)
    pltpu.sync_copy(x1_hbm, x1)
    pltpu.sync_copy(x2_hbm, x2)

    def chunk(c, carry):
        def group(k, c2):
            src = pl.ds(c * SB + k * 16, 16)
            dst = pl.ds(k * 16, 16)
            ir = idx_r[src]
            ic = idx_c[src]
            d0 = plsc.load_gather(x0, [ir]) - plsc.load_gather(x0, [ic])
            d1 = plsc.load_gather(x1, [ir]) - plsc.load_gather(x1, [ic])
            d2 = plsc.load_gather(x2, [ir]) - plsc.load_gather(x2, [ic])
            sr[dst] = d0 * d0 + d1 * d1 + d2 * d2
            s0[dst] = d0
            s1[dst] = d1
            s2[dst] = d2
            return c2
        lax.fori_loop(0, SB // 16, group, carry)
        off = pbase + c * SB
        pltpu.sync_copy(sr, flat4_hbm.at[pl.ds(0 * E + off, SB)])
        pltpu.sync_copy(s0, flat4_hbm.at[pl.ds(1 * E + off, SB)])
        pltpu.sync_copy(s1, flat4_hbm.at[pl.ds(2 * E + off, SB)])
        pltpu.sync_copy(s2, flat4_hbm.at[pl.ds(3 * E + off, SB)])
        return carry
    lax.fori_loop(0, EPW // SB, chunk, 0)

    _pipelined_gather(h_hbm, idxcat_hbm, hcat_hbm, idx_all,
                      (b0, b1), (sg0, sg1), (sw0, sw1), wid * CPW)


@functools.partial(
    pl.kernel, mesh=_mesh,
    out_type=jax.ShapeDtypeStruct((2 * E, D), jnp.float32),
    scratch_types=[
        pltpu.VMEM((CPW,), jnp.int32),
        pltpu.VMEM((GB, D), jnp.float32),
        pltpu.VMEM((GB, D), jnp.float32),
        pltpu.SemaphoreType.DMA,
        pltpu.SemaphoreType.DMA,
        pltpu.SemaphoreType.DMA,
        pltpu.SemaphoreType.DMA,
    ],
    compiler_params=_sc_params,
)
def _gather_h(h_hbm, idxcat_hbm, hcat_hbm,
              idx_all, b0, b1, sg0, sg1, sw0, sw1):
    _pipelined_gather(h_hbm, idxcat_hbm, hcat_hbm, idx_all,
                      (b0, b1), (sg0, sg1), (sw0, sw1), _worker_id() * CPW)


@functools.partial(
    pl.kernel, mesh=_mesh,
    out_type=jax.ShapeDtypeStruct((2 * N, D), jnp.float32),
    scratch_types=[
        pltpu.VMEM((GB,), jnp.int32),
        pltpu.VMEM((GB,), jnp.int32),
        pltpu.VMEM((GB, D), jnp.float32),
        pltpu.VMEM((GB, D), jnp.float32),
        pltpu.VMEM((RC, D), jnp.float32),
        pltpu.VMEM_SHARED((N, D), jnp.float32),
        pltpu.SemaphoreType.DMA,
        pltpu.SemaphoreType.DMA,
        pltpu.SemaphoreType.DMA,
        pltpu.SemaphoreType.DMA,
        pltpu.SemaphoreType.DMA,
        pltpu.SemaphoreType.DMA,
    ],
    compiler_params=_sc_params,
)
def _scatter_d(feat_hbm, row_hbm, out_hbm,
               i0, i1, f0, f1, stage, table,
               si0, si1, sf0, sf1, sa0, sa1):
    cid = lax.axis_index("c")
    sid = lax.axis_index("s")
    wid = sid * 2 + cid
    rbase = sid * RPT
    nchunk = jnp.where(sid == 15, (N - 15 * RPT) // RC, RPT // RC)

    def zrow(i, carry):
        def zcol(j, c2):
            stage[i, pl.ds(j * 16, 16)] = jnp.zeros((16,), jnp.float32)
            return c2
        return lax.fori_loop(0, D // 16, zcol, carry)
    lax.fori_loop(0, RC, zrow, 0)

    def ztab(k, carry):
        pltpu.sync_copy(stage, table.at[pl.ds(rbase + k * RC, RC)])
        return carry
    lax.fori_loop(0, nchunk, ztab, 0)
    plsc.subcore_barrier()

    base = wid * EPW
    ibufs = (i0, i1)
    fbufs = (f0, f1)
    sis = (si0, si1)
    sfs = (sf0, sf1)
    sas = (sa0, sa1)
    pltpu.async_copy(row_hbm.at[pl.ds(base, GB)], i0, si0)
    pltpu.async_copy(feat_hbm.at[pl.ds(base, GB)], f0, sf0)

    @pl.loop(0, NBS + 1, step=2)
    def _loop(g0):
        for b in range(2):
            g = g0 + b
            ib, fb, si, sf, sa = ibufs[b], fbufs[b], sis[b], sfs[b], sas[b]
            oib, ofb = ibufs[1 - b], fbufs[1 - b]
            osi, osf, osa = sis[1 - b], sfs[1 - b], sas[1 - b]

            @pl.when(g < NBS)
            def _():
                off = base + g * GB
                pltpu.make_async_copy(
                    row_hbm.at[pl.ds(off, GB)], ib, si).wait()
                pltpu.make_async_copy(
                    feat_hbm.at[pl.ds(off, GB)], fb, sf).wait()
                pltpu.async_copy(fb, table.at[ib], sa, add=True)

            @pl.when(g + 1 < NBS)
            def _():
                @pl.when(g >= 1)
                def _():
                    pltpu.make_async_copy(ofb, table.at[oib], osa).wait()
                noff = base + (g + 1) * GB
                pltpu.async_copy(row_hbm.at[pl.ds(noff, GB)], oib, osi)
                pltpu.async_copy(feat_hbm.at[pl.ds(noff, GB)], ofb, osf)

    pltpu.make_async_copy(f1, table.at[i1], sa1).wait()
    pltpu.make_async_copy(f0, table.at[i0], sa0).wait()
    plsc.subcore_barrier()

    def wb(k, carry):
        r = rbase + k * RC
        pltpu.sync_copy(table.at[pl.ds(r, RC)], stage)
        pltpu.sync_copy(stage, out_hbm.at[pl.ds(cid * N + r, RC)])
        return carry
    lax.fori_loop(0, nchunk, wb, 0)


def _edge_body(hr, hc, s, w1r, w1c, m, b1, w2, b2, aw, ab, feat):
    z = hr[...] @ w1r[...] + hc[...] @ w1c[...]
    z = z + lax.dot_general(s[...], m[...], (((0,), (0,)), ((), ())))
    z = jax.nn.silu(z + b1[...])
    mij = jax.nn.silu(z @ w2[...] + b2[...])
    att = jax.nn.sigmoid(mij @ aw[...] + ab[...])
    feat[...] = mij * att


def _equiv_body(hr, hc, s, w1r, w1c, m, b1, w2, b2, w3, wdx, wrad, trans):
    z = hr[...] @ w1r[...] + hc[...] @ w1c[...]
    z = z + lax.dot_general(s[...], m[...], (((0,), (0,)), ((), ())))
    z = jax.nn.silu(z + b1[...])
    z = jax.nn.silu(z @ w2[...] + b2[...])
    phi = z @ w3[...]
    dx = lax.dot_general(s[...], wdx[...], (((0,), (0,)), ((), ())))
    rad = lax.dot_general(s[...], wrad[...], (((0,), (0,)), ((), ())))
    cdn = dx / (jnp.sqrt(rad + 1e-8) + 1.0)
    trans[...] = cdn * phi


def _node_body(h, a0, a1, w1h, w1a, b1, w2, b2, out):
    agg = (a0[...] + a1[...]) * (1.0 / NORM_FACTOR)
    z = jax.nn.silu(h[...] @ w1h[...] + agg @ w1a[...] + b1[...])
    out[...] = h[...] + z @ w2[...] + b2[...]


def _xupd_body(x, p0, p1, out):
    upd = (p0[...] + p1[...])[:, 0:3] * (1.0 / NORM_FACTOR)
    out[...] = x[...] + upd


def _hrspec():
    return pl.BlockSpec((TE, D), lambda i: (i, 0))


def _hcspec():
    return pl.BlockSpec((TE, D), lambda i: (i + E // TE, 0))


def _espec(w):
    return pl.BlockSpec((TE, w), lambda i: (i, 0))


def _wspec(r, c):
    return pl.BlockSpec((r, c), lambda i: (0, 0))


def _sspec():
    return pl.BlockSpec((5, TE), lambda i: (0, i))


def _scal_mat(w1e):
    return jnp.concatenate(
        [w1e[0:1], jnp.zeros((3, D), jnp.float32), w1e[1:2]], axis=0)


def _edge_mlp(hcat, s, p):
    return pl.pallas_call(
        _edge_body,
        grid=(E // TE,),
        in_specs=[_hrspec(), _hcspec(), _sspec(),
                  _wspec(D, D), _wspec(D, D), _wspec(5, D), _wspec(1, D),
                  _wspec(D, D), _wspec(1, D), _wspec(D, 1), _wspec(1, 1)],
        out_specs=_espec(D),
        out_shape=jax.ShapeDtypeStruct((E, D), jnp.float32),
    )(hcat, hcat, s, p['e_w1'][0:D], p['e_w1'][D:2 * D],
      _scal_mat(p['e_w1'][2 * D:]), p['e_b1'].reshape(1, D),
      p['e_w2'], p['e_b2'].reshape(1, D), p['a_w'], p['a_b'].reshape(1, 1))


def _equiv_mlp(hcat, s, p):
    wdx = jnp.zeros((5, D), jnp.float32).at[1, 0].set(1.0) \
        .at[2, 1].set(1.0).at[3, 2].set(1.0)
    wrad = jnp.zeros((5, D), jnp.float32).at[0].set(1.0)
    return pl.pallas_call(
        _equiv_body,
        grid=(E // TE,),
        in_specs=[_hrspec(), _hcspec(), _sspec(),
                  _wspec(D, D), _wspec(D, D), _wspec(5, D), _wspec(1, D),
                  _wspec(D, D), _wspec(1, D), _wspec(D, 1),
                  _wspec(5, D), _wspec(5, D)],
        out_specs=_espec(D),
        out_shape=jax.ShapeDtypeStruct((E, D), jnp.float32),
    )(hcat, hcat, s, p['c_w1'][0:D], p['c_w1'][D:2 * D],
      _scal_mat(p['c_w1'][2 * D:]), p['c_b1'].reshape(1, D),
      p['c_w2'], p['c_b2'].reshape(1, D), p['c_w3'], wdx, wrad)


def _node_mlp(h, parts, p):
    nspec = pl.BlockSpec((TN, D), lambda i: (i, 0))
    return pl.pallas_call(
        _node_body,
        grid=(N // TN,),
        in_specs=[nspec,
                  pl.BlockSpec((TN, D), lambda i: (i, 0)),
                  pl.BlockSpec((TN, D), lambda i: (i + N // TN, 0)),
                  _wspec(D, D), _wspec(D, D), _wspec(1, D),
                  _wspec(D, D), _wspec(1, D)],
        out_specs=nspec,
        out_shape=jax.ShapeDtypeStruct((N, D), jnp.float32),
    )(h, parts, parts, p['n_w1'][0:D], p['n_w1'][D:2 * D],
      p['n_b1'].reshape(1, D), p['n_w2'], p['n_b2'].reshape(1, D))


def _x_update(x, xparts):
    nspec3 = pl.BlockSpec((TN, 3), lambda i: (i, 0))
    return pl.pallas_call(
        _xupd_body,
        grid=(N // TN,),
        in_specs=[nspec3,
                  pl.BlockSpec((TN, D), lambda i: (i, 0)),
                  pl.BlockSpec((TN, D), lambda i: (i + N // TN, 0))],
        out_specs=nspec3,
        out_shape=jax.ShapeDtypeStruct((N, 3), jnp.float32),
    )(x, xparts, xparts)


def kernel(h, x, edge_attr, params, edge_index):
    row = edge_index[0]
    idxcat = edge_index.reshape(2 * E)

    hcat, flat4 = _gather_hx(h, x[:, 0], x[:, 1], x[:, 2], idxcat)
    s = jnp.concatenate(
        [flat4.reshape(4, E), edge_attr.reshape(1, E)], axis=0)

    p0 = params['gcl_0']
    feat = _edge_mlp(hcat, s, p0)
    parts = _scatter_d(feat, row)
    h = _node_mlp(h, parts, p0)

    p1 = params['gcl_1']
    hcat = _gather_h(h, idxcat)
    feat = _edge_mlp(hcat, s, p1)
    parts = _scatter_d(feat, row)
    h = _node_mlp(h, parts, p1)

    pe = params['equiv']
    hcat = _gather_h(h, idxcat)
    trans = _equiv_mlp(hcat, s, pe)
    xparts = _scatter_d(trans, row)
    x = _x_update(x, xparts)

    return (h, x)
